# baseline (speedup 1.0000x reference)
import functools

import jax
import jax.numpy as jnp
from jax import lax
from jax.experimental import pallas as pl
from jax.experimental.pallas import tpu as pltpu
from jax.experimental.pallas import tpu_sc as plsc

_B = 1024
_V = 100000


_NC = 2
_NS = 16
_NW = _NC * _NS
_BPW = _B // _NW


def _sc_gather_body(xflat, tgt, out, tgt_v, idx_v, val_v, sem):
    c = lax.axis_index("c")
    s = lax.axis_index("s")
    wid = s * _NC + c
    base = wid * _BPW
    pltpu.sync_copy(tgt.at[pl.ds(base, _BPW)], tgt_v)
    for u in range(_BPW // 16):
        t = tgt_v[pl.ds(u * 16, 16)]
        rows = (base + u * 16) + lax.iota(jnp.int32, 16)
        idx_v[pl.ds(u * 16, 16)] = rows * _V + t
    pltpu.async_copy(xflat.at[idx_v], val_v, sem).wait()
    pltpu.sync_copy(val_v, out.at[pl.ds(base, _BPW)])


@functools.cache
def _sc_gather():
    return pl.kernel(
        _sc_gather_body,
        out_type=jax.ShapeDtypeStruct((_B,), jnp.float32),
        mesh=plsc.VectorSubcoreMesh(
            core_axis_name="c", subcore_axis_name="s",
            num_cores=_NC, num_subcores=_NS,
        ),
        scratch_types=[
            pltpu.VMEM((_BPW,), jnp.int32),
            pltpu.VMEM((_BPW,), jnp.int32),
            pltpu.VMEM((_BPW,), jnp.float32),
            pltpu.SemaphoreType.DMA,
        ],
    )


_BB = 256
_VC = 8192
_NBI = _B // _BB
_NVJ = (_V + _VC - 1) // _VC
_V_TAIL = _V - (_NVJ - 1) * _VC


def _tc_body(x_ref, tv_ref, out_ref, m_ref, s_ref):
    i = pl.program_id(0)
    j = pl.program_id(1)

    @pl.when(j == 0)
    def _init():
        m_ref[...] = jnp.full((_BB, 1), -jnp.inf, jnp.float32)
        s_ref[...] = jnp.zeros((_BB, 1), jnp.float32)

    def _update(x):
        bm = jnp.max(x, axis=1, keepdims=True)
        m_old = m_ref[...]
        m_new = jnp.maximum(m_old, bm)
        e_sum = jnp.sum(jnp.exp(x - m_new), axis=1, keepdims=True)
        s_new = s_ref[...] * jnp.exp(m_old - m_new) + e_sum
        s_ref[...] = s_new
        m_ref[...] = m_new
        return m_new, s_new

    @pl.when(j != _NVJ - 1)
    def _full():
        _update(x_ref[...])

    @pl.when(j == _NVJ - 1)
    def _fin():
        col = lax.broadcasted_iota(jnp.int32, (_BB, _VC), 1)
        xm = jnp.where(col < _V_TAIL, x_ref[...], -jnp.inf)
        m_new, s_new = _update(xm)
        lse = m_new + jnp.log(s_new)
        part = jnp.sum(lse - tv_ref[...]) * (1.0 / _B)

        @pl.when(i == 0)
        def _first():
            out_ref[0, 0] = part

        @pl.when(i != 0)
        def _rest():
            out_ref[0, 0] = out_ref[0, 0] + part


_tc_pass = pl.pallas_call(
    _tc_body,
    grid=(_NBI, _NVJ),
    in_specs=[
        pl.BlockSpec((_BB, _VC), lambda i, j: (i, j)),
        pl.BlockSpec((_BB, 1), lambda i, j: (i, 0)),
    ],
    out_specs=pl.BlockSpec((1, 1), lambda i, j: (0, 0), memory_space=pltpu.SMEM),
    out_shape=jax.ShapeDtypeStruct((1, 1), jnp.float32),
    scratch_shapes=[
        pltpu.VMEM((_BB, 1), jnp.float32),
        pltpu.VMEM((_BB, 1), jnp.float32),
    ],
    compiler_params=pltpu.CompilerParams(
        dimension_semantics=("arbitrary", "arbitrary"),
    ),
)


def kernel(inputs, targets):
    tvals = _sc_gather()(inputs.reshape(-1), targets.astype(jnp.int32))
    out = _tc_pass(inputs, tvals.reshape(_B, 1))
    return out[0, 0]


# speedup vs baseline: 1.8639x; 1.8639x over previous
import functools

import jax
import jax.numpy as jnp
from jax import lax
from jax.experimental import pallas as pl
from jax.experimental.pallas import tpu as pltpu
from jax.experimental.pallas import tpu_sc as plsc

_B = 1024
_V = 100000


_GPS = 16
_GSTEPS = _B // _GPS


def _gather_body(tgt_ref, *refs):
    out_ref = refs[-1]
    step = pl.program_id(0)
    ci = lax.broadcasted_iota(jnp.int32, (1, 128), 1)
    for q in range(_GPS):
        t = tgt_ref[step * _GPS + q]
        row = refs[q][pl.ds(q % 8, 1), :]
        v = jnp.sum(jnp.where(ci == lax.rem(t, 128), row, 0.0))
        out_ref[0, 0, q] = v


def _gather_in_spec(q):
    def idx(s, tgt_ref):
        samp = s * _GPS + q
        return (samp // 8, tgt_ref[samp] // 128)

    return pl.BlockSpec((8, 128), idx)


_tc_gather = pl.pallas_call(
    _gather_body,
    grid_spec=pltpu.PrefetchScalarGridSpec(
        num_scalar_prefetch=1,
        grid=(_GSTEPS,),
        in_specs=[_gather_in_spec(q) for q in range(_GPS)],
        out_specs=pl.BlockSpec(
            (1, 1, _GPS), lambda s, tgt_ref: (s, 0, 0), memory_space=pltpu.SMEM
        ),
    ),
    out_shape=jax.ShapeDtypeStruct((_GSTEPS, 1, _GPS), jnp.float32),
    compiler_params=pltpu.CompilerParams(
        dimension_semantics=("arbitrary",),
    ),
)


_BB = 256
_VC = 8192
_NBI = _B // _BB
_NVJ = (_V + _VC - 1) // _VC
_V_TAIL = _V - (_NVJ - 1) * _VC


def _tc_body(x_ref, lse_ref, m_ref, s_ref):
    j = pl.program_id(1)

    @pl.when(j == 0)
    def _init():
        m_ref[...] = jnp.full((_BB, 1), -jnp.inf, jnp.float32)
        s_ref[...] = jnp.zeros((_BB, 1), jnp.float32)

    def _update(x):
        bm = jnp.max(x, axis=1, keepdims=True)
        m_old = m_ref[...]
        m_new = jnp.maximum(m_old, bm)
        e_sum = jnp.sum(jnp.exp(x - m_new), axis=1, keepdims=True)
        s_new = s_ref[...] * jnp.exp(m_old - m_new) + e_sum
        s_ref[...] = s_new
        m_ref[...] = m_new
        return m_new, s_new

    @pl.when(j != _NVJ - 1)
    def _full():
        _update(x_ref[...])

    @pl.when(j == _NVJ - 1)
    def _fin():
        col = lax.broadcasted_iota(jnp.int32, (_BB, _VC), 1)
        xm = jnp.where(col < _V_TAIL, x_ref[...], -jnp.inf)
        m_new, s_new = _update(xm)
        lse_ref[...] = m_new + jnp.log(s_new)


_tc_lse = pl.pallas_call(
    _tc_body,
    grid=(_NBI, _NVJ),
    in_specs=[pl.BlockSpec((_BB, _VC), lambda i, j: (i, j))],
    out_specs=pl.BlockSpec((_BB, 1), lambda i, j: (i, 0)),
    out_shape=jax.ShapeDtypeStruct((_B, 1), jnp.float32),
    scratch_shapes=[
        pltpu.VMEM((_BB, 1), jnp.float32),
        pltpu.VMEM((_BB, 1), jnp.float32),
    ],
    compiler_params=pltpu.CompilerParams(
        dimension_semantics=("arbitrary", "arbitrary"),
    ),
)


_NC = 2
_NS = 16


def _sc_mine_body(lse, tv, out, lse_v, tv_v, out_v):
    c = lax.axis_index("c")
    s = lax.axis_index("s")
    wid = s * _NC + c

    @pl.when(wid == 0)
    def _():
        pltpu.sync_copy(lse, lse_v)
        pltpu.sync_copy(tv, tv_v)
        acc = jnp.zeros((16,), jnp.float32)
        for u in range(_B // 16):
            acc = acc + (lse_v[pl.ds(u * 16, 16)] - tv_v[pl.ds(u * 16, 16)])
        total = lax.reduce_sum_p.bind(acc, axes=(0,))
        out_v[...] = jnp.full((16,), total * (1.0 / _B), jnp.float32)
        pltpu.sync_copy(out_v, out)


@functools.cache
def _sc_mine():
    return pl.kernel(
        _sc_mine_body,
        out_type=jax.ShapeDtypeStruct((16,), jnp.float32),
        mesh=plsc.VectorSubcoreMesh(
            core_axis_name="c", subcore_axis_name="s",
            num_cores=_NC, num_subcores=_NS,
        ),
        scratch_types=[
            pltpu.VMEM((_B,), jnp.float32),
            pltpu.VMEM((_B,), jnp.float32),
            pltpu.VMEM((16,), jnp.float32),
        ],
        compiler_params=pltpu.CompilerParams(needs_layout_passes=False),
    )


def kernel(inputs, targets):
    tgt = targets.astype(jnp.int32)
    tv = _tc_gather(tgt, *([inputs] * _GPS))
    lse = _tc_lse(inputs)
    loss = _sc_mine()(lse.reshape(_B), tv.reshape(_B))
    return loss[0]


# speedup vs baseline: 1.8937x; 1.0160x over previous
import functools

import jax
import jax.numpy as jnp
from jax import lax
from jax.experimental import pallas as pl
from jax.experimental.pallas import tpu as pltpu
from jax.experimental.pallas import tpu_sc as plsc

_B = 1024
_V = 100000


_DEPTH = 16


def _gather_body(tgt_ref, tgtv_ref, x_any, out_ref, win, sems):
    def mk(k):
        cb = pl.multiple_of((tgt_ref[k] // 128) * 128, 128)
        return pltpu.make_async_copy(
            x_any.at[pl.ds(k, 1), pl.ds(cb, 128)],
            win.at[pl.ds(k, 1), :],
            sems.at[lax.rem(k, _DEPTH)],
        )

    def issue(k, carry):
        @pl.when(k >= _DEPTH)
        def _():
            mk(k - _DEPTH).wait()

        mk(k).start()
        return carry

    lax.fori_loop(0, _B, issue, 0, unroll=False)
    for d in range(_DEPTH):
        mk(_B - _DEPTH + d).wait()
    cols = lax.broadcasted_iota(jnp.int32, (_B, 128), 1)
    sel = cols == lax.rem(tgtv_ref[...], 128)
    out_ref[...] = jnp.sum(
        jnp.where(sel, win[...], 0.0), axis=1, keepdims=True
    )


_tc_gather = pl.pallas_call(
    _gather_body,
    grid_spec=pltpu.PrefetchScalarGridSpec(
        num_scalar_prefetch=1,
        grid=(1,),
        in_specs=[
            pl.BlockSpec((_B, 1), lambda s, tgt_ref: (0, 0)),
            pl.BlockSpec(memory_space=pl.ANY),
        ],
        out_specs=pl.BlockSpec((_B, 1), lambda s, tgt_ref: (0, 0)),
        scratch_shapes=[
            pltpu.VMEM((_B, 128), jnp.float32),
            pltpu.SemaphoreType.DMA((_DEPTH,)),
        ],
    ),
    out_shape=jax.ShapeDtypeStruct((_B, 1), jnp.float32),
    compiler_params=pltpu.CompilerParams(
        dimension_semantics=("arbitrary",),
    ),
)


_BB = 256
_VC = 8192
_NBI = _B // _BB
_NVJ = (_V + _VC - 1) // _VC
_V_TAIL = _V - (_NVJ - 1) * _VC


def _tc_body(x_ref, lse_ref, m_ref, s_ref):
    j = pl.program_id(1)

    @pl.when(j == 0)
    def _init():
        m_ref[...] = jnp.full((_BB, 1), -jnp.inf, jnp.float32)
        s_ref[...] = jnp.zeros((_BB, 1), jnp.float32)

    def _update(x):
        bm = jnp.max(x, axis=1, keepdims=True)
        m_old = m_ref[...]
        m_new = jnp.maximum(m_old, bm)
        e_sum = jnp.sum(jnp.exp(x - m_new), axis=1, keepdims=True)
        s_new = s_ref[...] * jnp.exp(m_old - m_new) + e_sum
        s_ref[...] = s_new
        m_ref[...] = m_new
        return m_new, s_new

    @pl.when(j != _NVJ - 1)
    def _full():
        _update(x_ref[...])

    @pl.when(j == _NVJ - 1)
    def _fin():
        col = lax.broadcasted_iota(jnp.int32, (_BB, _VC), 1)
        xm = jnp.where(col < _V_TAIL, x_ref[...], -jnp.inf)
        m_new, s_new = _update(xm)
        lse_ref[...] = m_new + jnp.log(s_new)


_tc_lse = pl.pallas_call(
    _tc_body,
    grid=(_NBI, _NVJ),
    in_specs=[pl.BlockSpec((_BB, _VC), lambda i, j: (i, j))],
    out_specs=pl.BlockSpec((_BB, 1), lambda i, j: (i, 0)),
    out_shape=jax.ShapeDtypeStruct((_B, 1), jnp.float32),
    scratch_shapes=[
        pltpu.VMEM((_BB, 1), jnp.float32),
        pltpu.VMEM((_BB, 1), jnp.float32),
    ],
    compiler_params=pltpu.CompilerParams(
        dimension_semantics=("arbitrary", "arbitrary"),
    ),
)


_NC = 2
_NS = 16


def _sc_mine_body(lse, tv, out, lse_v, tv_v, out_v):
    c = lax.axis_index("c")
    s = lax.axis_index("s")
    wid = s * _NC + c

    @pl.when(wid == 0)
    def _():
        pltpu.sync_copy(lse, lse_v)
        pltpu.sync_copy(tv, tv_v)
        acc = jnp.zeros((16,), jnp.float32)
        for u in range(_B // 16):
            acc = acc + (lse_v[pl.ds(u * 16, 16)] - tv_v[pl.ds(u * 16, 16)])
        total = lax.reduce_sum_p.bind(acc, axes=(0,))
        out_v[...] = jnp.full((16,), total * (1.0 / _B), jnp.float32)
        pltpu.sync_copy(out_v, out)


@functools.cache
def _sc_mine():
    return pl.kernel(
        _sc_mine_body,
        out_type=jax.ShapeDtypeStruct((16,), jnp.float32),
        mesh=plsc.VectorSubcoreMesh(
            core_axis_name="c", subcore_axis_name="s",
            num_cores=_NC, num_subcores=_NS,
        ),
        scratch_types=[
            pltpu.VMEM((_B,), jnp.float32),
            pltpu.VMEM((_B,), jnp.float32),
            pltpu.VMEM((16,), jnp.float32),
        ],
        compiler_params=pltpu.CompilerParams(needs_layout_passes=False),
    )


def kernel(inputs, targets):
    tgt = targets.astype(jnp.int32)
    tv = _tc_gather(tgt, tgt.reshape(_B, 1), inputs)
    lse = _tc_lse(inputs)
    loss = _sc_mine()(lse.reshape(_B), tv.reshape(_B))
    return loss[0]


# speedup vs baseline: 7.2492x; 3.8280x over previous
import functools

import jax
import jax.numpy as jnp
from jax import lax
from jax.experimental import pallas as pl
from jax.experimental.pallas import tpu as pltpu
from jax.experimental.pallas import tpu_sc as plsc

_B = 1024
_V = 100000


_VR = 4000
_NVJ = _V // _VR
_GPB = -(-_B // _NVJ)


def _tc_body(tgt_ref, x_ref, x_any, lse_ref, tv_ref, m_ref, s_ref, win, gsem):
    j = pl.program_id(0)

    @pl.when(j == 0)
    def _init():
        m_ref[...] = jnp.full((1, _B), -jnp.inf, jnp.float32)
        s_ref[...] = jnp.zeros((1, _B), jnp.float32)

    for q in range(_GPB):
        k = j * _GPB + q

        @pl.when(k < _B)
        def _():
            cb = pl.multiple_of((k // 128) * 128, 128)
            pltpu.make_async_copy(
                x_any.at[pl.ds(tgt_ref[k], 1), pl.ds(cb, 128)],
                win.at[pl.ds(k, 1), :],
                gsem,
            ).start()

    x = x_ref[...]
    bm = jnp.max(x, axis=0, keepdims=True)
    m_old = m_ref[...]
    m_new = jnp.maximum(m_old, bm)
    e_sum = jnp.sum(jnp.exp(x - m_new), axis=0, keepdims=True)
    s_new = s_ref[...] * jnp.exp(m_old - m_new) + e_sum
    s_ref[...] = s_new
    m_ref[...] = m_new

    @pl.when(j == _NVJ - 1)
    def _fin():
        lse_ref[...] = m_new + jnp.log(s_new)
        pltpu.make_async_copy(
            x_any.at[pl.ds(0, _B), pl.ds(0, 128)], win, gsem
        ).wait()
        rows = lax.broadcasted_iota(jnp.int32, (_B, 128), 0)
        cols = lax.broadcasted_iota(jnp.int32, (_B, 128), 1)
        sel = cols == lax.rem(rows, 128)
        tv_ref[...] = jnp.sum(
            jnp.where(sel, win[...], 0.0), axis=1, keepdims=True
        )


_tc_main = pl.pallas_call(
    _tc_body,
    grid_spec=pltpu.PrefetchScalarGridSpec(
        num_scalar_prefetch=1,
        grid=(_NVJ,),
        in_specs=[
            pl.BlockSpec((_VR, _B), lambda j, tgt_ref: (j, 0)),
            pl.BlockSpec(memory_space=pl.ANY),
        ],
        out_specs=[
            pl.BlockSpec((1, _B), lambda j, tgt_ref: (0, 0)),
            pl.BlockSpec((_B, 1), lambda j, tgt_ref: (0, 0)),
        ],
        scratch_shapes=[
            pltpu.VMEM((1, _B), jnp.float32),
            pltpu.VMEM((1, _B), jnp.float32),
            pltpu.VMEM((_B, 128), jnp.float32),
            pltpu.SemaphoreType.DMA,
        ],
    ),
    out_shape=[
        jax.ShapeDtypeStruct((1, _B), jnp.float32),
        jax.ShapeDtypeStruct((_B, 1), jnp.float32),
    ],
    compiler_params=pltpu.CompilerParams(
        dimension_semantics=("arbitrary",),
        vmem_limit_bytes=57 * 1024 * 1024,
    ),
)


_NC = 2
_NS = 16


def _sc_mine_body(lse, tv, out, lse_v, tv_v, out_v):
    c = lax.axis_index("c")
    s = lax.axis_index("s")
    wid = s * _NC + c

    @pl.when(wid == 0)
    def _():
        pltpu.sync_copy(lse, lse_v)
        pltpu.sync_copy(tv, tv_v)
        acc = jnp.zeros((16,), jnp.float32)
        for u in range(_B // 16):
            acc = acc + (lse_v[pl.ds(u * 16, 16)] - tv_v[pl.ds(u * 16, 16)])
        total = lax.reduce_sum_p.bind(acc, axes=(0,))
        out_v[...] = jnp.full((16,), total * (1.0 / _B), jnp.float32)
        pltpu.sync_copy(out_v, out)


@functools.cache
def _sc_mine():
    return pl.kernel(
        _sc_mine_body,
        out_type=jax.ShapeDtypeStruct((16,), jnp.float32),
        mesh=plsc.VectorSubcoreMesh(
            core_axis_name="c", subcore_axis_name="s",
            num_cores=_NC, num_subcores=_NS,
        ),
        scratch_types=[
            pltpu.VMEM((_B,), jnp.float32),
            pltpu.VMEM((_B,), jnp.float32),
            pltpu.VMEM((16,), jnp.float32),
        ],
        compiler_params=pltpu.CompilerParams(needs_layout_passes=False),
    )


def kernel(inputs, targets):
    xt = inputs.T
    tgt = targets.astype(jnp.int32)
    lse, tv = _tc_main(tgt, xt, xt)
    loss = _sc_mine()(lse.reshape(_B), tv.reshape(_B))
    return loss[0]
